# baseline (speedup 1.0000x reference)
import functools
import math

import jax
import jax.numpy as jnp
from jax import lax
from jax.experimental import pallas as pl
from jax.experimental.pallas import tpu as pltpu
from jax.experimental.pallas import tpu_sc as plsc

SUPPORT_NUM = 1
NEIGHBOR_NUM = 20
FEATURE_SIZE = 1024
MAX_SAMPLES = 256
CLUSTER_SIZE = 64
BN_EPS = 1e-5


def _l2norm(x, axis):
    n = jnp.sqrt(jnp.sum(x * x, axis=axis, keepdims=True))
    return x / jnp.maximum(n, 1e-12)


def _knn_body(vr_ref, vt_ref, out_ref, *, v, n_extract, rows):
    c = v // 128
    vr = vr_ref[0]
    vt = vt_ref[0]
    inner = jnp.dot(vr, vt, preferred_element_type=jnp.float32)
    quad_r = jnp.sum(vr * vr, axis=1, keepdims=True)
    quad_t = jnp.sum(vt * vt, axis=0, keepdims=True)
    dist = (-2.0 * inner + quad_r + quad_t).reshape(rows, c, 128)
    ic = jax.lax.broadcasted_iota(jnp.int32, (1, c, 128), 1)
    il = jax.lax.broadcasted_iota(jnp.int32, (1, c, 128), 2)
    iota_f = (ic * 128 + il).astype(jnp.float32)
    big = jnp.float32(3.0e38)
    iota_o = jax.lax.broadcasted_iota(jnp.int32, (1, 32), 1)
    acc = jnp.zeros((rows, 32), jnp.float32)
    for j in range(n_extract):
        m = jnp.min(jnp.min(dist, axis=2), axis=1, keepdims=True)
        mi = jnp.where(dist == m[:, :, None], iota_f, big)
        ji = jnp.min(jnp.min(mi, axis=2), axis=1, keepdims=True)
        dist = jnp.where(mi == ji[:, :, None], big, dist)
        acc = jnp.where(iota_o == j, ji, acc)
    out_ref[0] = acc.astype(jnp.int32)


def _knn_index(vertices, neighbor_num):
    bs, v, _ = vertices.shape
    rows = min(v, 256)
    vt = jnp.transpose(vertices, (0, 2, 1))
    out = pl.pallas_call(
        functools.partial(_knn_body, v=v, n_extract=neighbor_num + 1,
                          rows=rows),
        grid=(bs, v // rows),
        in_specs=[
            pl.BlockSpec((1, rows, 3), lambda b, i: (b, i, 0)),
            pl.BlockSpec((1, 3, v), lambda b, i: (b, 0, 0)),
        ],
        out_specs=pl.BlockSpec((1, rows, 32), lambda b, i: (b, i, 0)),
        out_shape=jax.ShapeDtypeStruct((bs, v, 32), jnp.int32),
    )(vertices, vt)
    return out[:, :, 1:neighbor_num + 1]


_NW = 32


def _sc_gather(table, idx):
    t, d = table.shape
    n = idx.shape[0]
    assert d % 16 == 0 and n % (8 * _NW) == 0
    b_per_w = n // _NW
    chunk = min(b_per_w, max(8, min(128, (98304 // d) & ~7)))
    while b_per_w % chunk:
        chunk -= 8
    n_chunks = b_per_w // chunk
    mesh = plsc.VectorSubcoreMesh(core_axis_name="c", subcore_axis_name="s")

    @functools.partial(
        pl.kernel, mesh=mesh,
        out_type=jax.ShapeDtypeStruct((n, d), jnp.float32),
        scratch_types=[
            pltpu.VMEM((chunk,), jnp.int32),
            pltpu.VMEM((chunk, d), jnp.float32),
            pltpu.SemaphoreType.DMA,
        ],
        compiler_params=pltpu.CompilerParams(
            use_tc_tiling_on_sc=(d % 128 == 0)),
    )
    def gk(table_hbm, idx_hbm, out_hbm, idx_v, rows_v, sem):
        wid = lax.axis_index("s") * 2 + lax.axis_index("c")
        base = wid * b_per_w

        def body(i, carry):
            off = base + i * chunk
            pltpu.sync_copy(idx_hbm.at[pl.ds(off, chunk)], idx_v)
            pltpu.async_copy(table_hbm.at[idx_v], rows_v, sem).wait()
            pltpu.sync_copy(rows_v, out_hbm.at[pl.ds(off, chunk)])
            return carry

        lax.fori_loop(0, n_chunks, body, 0)

    return gk(table, idx)


def _gather_nbr(tensor, index):
    bs, v, d = tensor.shape
    _, m, n = index.shape
    dp = (d + 15) & ~15
    tab = tensor if d == dp else jnp.pad(tensor, ((0, 0), (0, 0), (0, dp - d)))
    flat_idx = (index + (jnp.arange(bs, dtype=index.dtype)[:, None, None] * v)
                ).reshape(-1)
    out = _sc_gather(tab.reshape(bs * v, dp), flat_idx)
    out = out.reshape(bs, m, n, dp)
    return out if d == dp else out[..., :d]


def _nbr_dir_norm(vertices, neighbor_index):
    neighbors = _gather_nbr(vertices, neighbor_index)
    direction = neighbors - vertices[:, :, None, :]
    return _l2norm(direction, axis=-1)


def _conv_surface(neighbor_index, nd, directions, kernel_num, support_num):
    bs, v, n = neighbor_index.shape
    sd = _l2norm(directions, axis=0)
    theta = jax.nn.relu(jnp.einsum('bvnd,dk->bvnk', nd, sd))
    theta = theta.reshape(bs, v, n, support_num, kernel_num)
    return jnp.sum(jnp.max(theta, axis=2), axis=2)


def _conv_layer(neighbor_index, nd, feature_map, weights, bias, directions, out_channel, support_num):
    bs, v, n = neighbor_index.shape
    sd = _l2norm(directions, axis=0)
    theta = jax.nn.relu(jnp.einsum('bvnd,dk->bvnk', nd, sd))
    feature_out = feature_map @ weights + bias
    feature_center = feature_out[:, :, :out_channel]
    feature_support = feature_out[:, :, out_channel:]
    feature_support = _gather_nbr(feature_support, neighbor_index)
    activation_support = (theta * feature_support).reshape(bs, v, n, support_num, out_channel)
    activation_support = jnp.sum(jnp.max(activation_support, axis=2), axis=2)
    return feature_center + activation_support


def _pool_layer(vertices, feature_map, neighbor_index, pooling_rate):
    bs, v, _ = vertices.shape
    pool_num = v // pooling_rate
    sample_idx = jnp.arange(pool_num) * pooling_rate
    ni_s = neighbor_index[:, sample_idx, :]
    pooled = jnp.max(_gather_nbr(feature_map, ni_s), axis=2)
    return vertices[:, sample_idx, :], pooled


def _batchnorm(x, gamma, beta):
    mean = jnp.mean(x, axis=0)
    var = jnp.mean((x - mean) ** 2, axis=0)
    return gamma * (x - mean) / jnp.sqrt(var + BN_EPS) + beta


def _hidden_mm_body(x_ref, w_ref, o_ref):
    @pl.when(pl.program_id(0) == 0)
    def _init():
        o_ref[...] = jnp.zeros_like(o_ref)

    o_ref[...] += jnp.dot(x_ref[...], w_ref[...],
                          preferred_element_type=jnp.float32)


def _hidden_matmul(xv, w):
    b = xv.shape[0]
    k, n = w.shape
    xp = jnp.zeros((8, k), xv.dtype).at[:b].set(xv)
    kb = 4096
    out = pl.pallas_call(
        _hidden_mm_body,
        grid=(k // kb,),
        in_specs=[
            pl.BlockSpec((8, kb), lambda i: (0, i)),
            pl.BlockSpec((kb, n), lambda i: (i, 0)),
        ],
        out_specs=pl.BlockSpec((8, n), lambda i: (0, 0)),
        out_shape=jax.ShapeDtypeStruct((8, n), jnp.float32),
        compiler_params=pltpu.CompilerParams(
            dimension_semantics=("arbitrary",)),
    )(xp, w)
    return out[:b]


def kernel(x, dirs0, w1, b1, dirs1, w2, b2, dirs2, w3, b3, dirs3, w4, b4, dirs4, cluster_w, cluster_w2, hidden1_w, bn1_g, bn1_b, bn2_g, bn2_b, gating_w, gbn_g, gbn_b, gem_p):
    bs = x.shape[0]
    vertices = x.reshape(bs, -1, 3)
    ni = _knn_index(vertices, NEIGHBOR_NUM)
    nd = _nbr_dir_norm(vertices, ni)
    fm0 = jax.nn.relu(_conv_surface(ni, nd, dirs0, 32, SUPPORT_NUM))
    fm1 = jax.nn.relu(_conv_layer(ni, nd, fm0, w1, b1, dirs1, 64, SUPPORT_NUM))
    vertices, fm1 = _pool_layer(vertices, fm1, ni, 4)
    ni = _knn_index(vertices, NEIGHBOR_NUM)
    nd = _nbr_dir_norm(vertices, ni)
    fm2 = jax.nn.relu(_conv_layer(ni, nd, fm1, w2, b2, dirs2, 128, SUPPORT_NUM))
    fm3 = jax.nn.relu(_conv_layer(ni, nd, fm2, w3, b3, dirs3, 256, SUPPORT_NUM))
    vertices, fm3 = _pool_layer(vertices, fm3, ni, 4)
    ni = _knn_index(vertices, NEIGHBOR_NUM)
    nd = _nbr_dir_norm(vertices, ni)
    fm4 = _conv_layer(ni, nd, fm3, w4, b4, dirs4, 1024, SUPPORT_NUM)

    p = gem_p[0]
    g = jnp.maximum(fm4, 1e-06) ** p
    y = jnp.mean(g, axis=1) ** (1.0 / p)

    xv = fm4
    activation = jnp.einsum('bnf,fc->bnc', xv, cluster_w)
    activation = _batchnorm(activation.reshape(-1, CLUSTER_SIZE), bn1_g, bn1_b)
    activation = jax.nn.softmax(activation.reshape(-1, MAX_SAMPLES, CLUSTER_SIZE), axis=-1)
    a_sum = jnp.sum(activation, axis=-2, keepdims=True)
    a = a_sum * cluster_w2
    vlad = jnp.einsum('bnc,bnf->bfc', activation, xv)
    vlad = vlad - a
    vlad = _l2norm(vlad, axis=1)
    vlad = vlad.reshape(-1, CLUSTER_SIZE * FEATURE_SIZE)
    vlad = _l2norm(vlad, axis=1)
    vlad = _batchnorm(_hidden_matmul(vlad, hidden1_w), bn2_g, bn2_b)
    gates = jax.nn.sigmoid(_batchnorm(vlad @ gating_w, gbn_g, gbn_b))
    return (y, vlad * gates)


# speedup vs baseline: 1.8366x; 1.8366x over previous
import functools
import math

import jax
import jax.numpy as jnp
from jax import lax
from jax.experimental import pallas as pl
from jax.experimental.pallas import tpu as pltpu
from jax.experimental.pallas import tpu_sc as plsc

SUPPORT_NUM = 1
NEIGHBOR_NUM = 20
FEATURE_SIZE = 1024
MAX_SAMPLES = 256
CLUSTER_SIZE = 64
BN_EPS = 1e-5


def _l2norm(x, axis):
    n = jnp.sqrt(jnp.sum(x * x, axis=axis, keepdims=True))
    return x / jnp.maximum(n, 1e-12)


def _knn_body(vr_ref, vt_ref, out_ref, *, v, n_extract, rows):
    c = v // 128
    vr = vr_ref[0]
    vt = vt_ref[0]
    inner = jnp.dot(vr, vt, preferred_element_type=jnp.float32)
    quad_r = jnp.sum(vr * vr, axis=1, keepdims=True)
    quad_t = jnp.sum(vt * vt, axis=0, keepdims=True)
    dist = -2.0 * inner + quad_r + quad_t
    iota_f = jax.lax.broadcasted_iota(
        jnp.int32, (1, v), 1).astype(jnp.float32)
    big = jnp.float32(3.0e38)
    iota_o = jax.lax.broadcasted_iota(jnp.int32, (1, 32), 1)
    acc = jnp.zeros((rows, 32), jnp.float32)
    for j in range(n_extract):
        m = jnp.min(dist, axis=1, keepdims=True)
        mi = jnp.where(dist == m, iota_f, big)
        ji = jnp.min(mi, axis=1, keepdims=True)
        dist = jnp.where(mi == ji, big, dist)
        acc = jnp.where(iota_o == j, ji, acc)
    out_ref[0] = acc.astype(jnp.int32)


def _knn_index(vertices, neighbor_num):
    bs, v, _ = vertices.shape
    rows = min(v, 256)
    vt = jnp.transpose(vertices, (0, 2, 1))
    out = pl.pallas_call(
        functools.partial(_knn_body, v=v, n_extract=neighbor_num + 1,
                          rows=rows),
        grid=(bs, v // rows),
        in_specs=[
            pl.BlockSpec((1, rows, 3), lambda b, i: (b, i, 0)),
            pl.BlockSpec((1, 3, v), lambda b, i: (b, 0, 0)),
        ],
        out_specs=pl.BlockSpec((1, rows, 32), lambda b, i: (b, i, 0)),
        out_shape=jax.ShapeDtypeStruct((bs, v, 32), jnp.int32),
    )(vertices, vt)
    return out[:, :, 1:neighbor_num + 1]


_NW = 32


def _sc_gather(table, idx):
    t, d = table.shape
    n = idx.shape[0]
    assert d % 16 == 0 and n % (8 * _NW) == 0
    b_per_w = n // _NW
    chunk = min(b_per_w, max(8, min(128, (98304 // d) & ~7)))
    while b_per_w % chunk:
        chunk -= 8
    n_chunks = b_per_w // chunk
    mesh = plsc.VectorSubcoreMesh(core_axis_name="c", subcore_axis_name="s")

    @functools.partial(
        pl.kernel, mesh=mesh,
        out_type=jax.ShapeDtypeStruct((n, d), jnp.float32),
        scratch_types=[
            pltpu.VMEM((chunk,), jnp.int32),
            pltpu.VMEM((chunk, d), jnp.float32),
            pltpu.SemaphoreType.DMA,
        ],
        compiler_params=pltpu.CompilerParams(
            use_tc_tiling_on_sc=(d % 128 == 0)),
    )
    def gk(table_hbm, idx_hbm, out_hbm, idx_v, rows_v, sem):
        wid = lax.axis_index("s") * 2 + lax.axis_index("c")
        base = wid * b_per_w

        def body(i, carry):
            off = base + i * chunk
            pltpu.sync_copy(idx_hbm.at[pl.ds(off, chunk)], idx_v)
            pltpu.async_copy(table_hbm.at[idx_v], rows_v, sem).wait()
            pltpu.sync_copy(rows_v, out_hbm.at[pl.ds(off, chunk)])
            return carry

        lax.fori_loop(0, n_chunks, body, 0)

    return gk(table, idx)


def _gather_nbr(tensor, index):
    bs, v, d = tensor.shape
    _, m, n = index.shape
    dp = (d + 15) & ~15
    tab = tensor if d == dp else jnp.pad(tensor, ((0, 0), (0, 0), (0, dp - d)))
    flat_idx = (index + (jnp.arange(bs, dtype=index.dtype)[:, None, None] * v)
                ).reshape(-1)
    out = _sc_gather(tab.reshape(bs * v, dp), flat_idx)
    out = out.reshape(bs, m, n, dp)
    return out if d == dp else out[..., :d]


def _nbr_dir_norm(vertices, neighbor_index):
    neighbors = _gather_nbr(vertices, neighbor_index)
    direction = neighbors - vertices[:, :, None, :]
    return _l2norm(direction, axis=-1)


def _conv_surface(neighbor_index, nd, directions, kernel_num, support_num):
    bs, v, n = neighbor_index.shape
    sd = _l2norm(directions, axis=0)
    theta = jax.nn.relu(jnp.einsum('bvnd,dk->bvnk', nd, sd))
    theta = theta.reshape(bs, v, n, support_num, kernel_num)
    return jnp.sum(jnp.max(theta, axis=2), axis=2)


def _conv_layer(neighbor_index, nd, feature_map, weights, bias, directions, out_channel, support_num):
    bs, v, n = neighbor_index.shape
    sd = _l2norm(directions, axis=0)
    theta = jax.nn.relu(jnp.einsum('bvnd,dk->bvnk', nd, sd))
    feature_out = feature_map @ weights + bias
    feature_center = feature_out[:, :, :out_channel]
    feature_support = feature_out[:, :, out_channel:]
    feature_support = _gather_nbr(feature_support, neighbor_index)
    activation_support = (theta * feature_support).reshape(bs, v, n, support_num, out_channel)
    activation_support = jnp.sum(jnp.max(activation_support, axis=2), axis=2)
    return feature_center + activation_support


def _pool_layer(vertices, feature_map, neighbor_index, pooling_rate):
    bs, v, _ = vertices.shape
    pool_num = v // pooling_rate
    sample_idx = jnp.arange(pool_num) * pooling_rate
    ni_s = neighbor_index[:, sample_idx, :]
    pooled = jnp.max(_gather_nbr(feature_map, ni_s), axis=2)
    return vertices[:, sample_idx, :], pooled


def _batchnorm(x, gamma, beta):
    mean = jnp.mean(x, axis=0)
    var = jnp.mean((x - mean) ** 2, axis=0)
    return gamma * (x - mean) / jnp.sqrt(var + BN_EPS) + beta


def _hidden_mm_body(x_ref, w_ref, o_ref):
    @pl.when(pl.program_id(0) == 0)
    def _init():
        o_ref[...] = jnp.zeros_like(o_ref)

    o_ref[...] += jnp.dot(x_ref[...], w_ref[...],
                          preferred_element_type=jnp.float32)


def _hidden_matmul(xv, w):
    b = xv.shape[0]
    k, n = w.shape
    xp = jnp.zeros((8, k), xv.dtype).at[:b].set(xv)
    kb = 4096
    out = pl.pallas_call(
        _hidden_mm_body,
        grid=(k // kb,),
        in_specs=[
            pl.BlockSpec((8, kb), lambda i: (0, i)),
            pl.BlockSpec((kb, n), lambda i: (i, 0)),
        ],
        out_specs=pl.BlockSpec((8, n), lambda i: (0, 0)),
        out_shape=jax.ShapeDtypeStruct((8, n), jnp.float32),
        compiler_params=pltpu.CompilerParams(
            dimension_semantics=("arbitrary",)),
    )(xp, w)
    return out[:b]


def kernel(x, dirs0, w1, b1, dirs1, w2, b2, dirs2, w3, b3, dirs3, w4, b4, dirs4, cluster_w, cluster_w2, hidden1_w, bn1_g, bn1_b, bn2_g, bn2_b, gating_w, gbn_g, gbn_b, gem_p):
    bs = x.shape[0]
    vertices = x.reshape(bs, -1, 3)
    ni = _knn_index(vertices, NEIGHBOR_NUM)
    nd = _nbr_dir_norm(vertices, ni)
    fm0 = jax.nn.relu(_conv_surface(ni, nd, dirs0, 32, SUPPORT_NUM))
    fm1 = jax.nn.relu(_conv_layer(ni, nd, fm0, w1, b1, dirs1, 64, SUPPORT_NUM))
    vertices, fm1 = _pool_layer(vertices, fm1, ni, 4)
    ni = _knn_index(vertices, NEIGHBOR_NUM)
    nd = _nbr_dir_norm(vertices, ni)
    fm2 = jax.nn.relu(_conv_layer(ni, nd, fm1, w2, b2, dirs2, 128, SUPPORT_NUM))
    fm3 = jax.nn.relu(_conv_layer(ni, nd, fm2, w3, b3, dirs3, 256, SUPPORT_NUM))
    vertices, fm3 = _pool_layer(vertices, fm3, ni, 4)
    ni = _knn_index(vertices, NEIGHBOR_NUM)
    nd = _nbr_dir_norm(vertices, ni)
    fm4 = _conv_layer(ni, nd, fm3, w4, b4, dirs4, 1024, SUPPORT_NUM)

    p = gem_p[0]
    g = jnp.maximum(fm4, 1e-06) ** p
    y = jnp.mean(g, axis=1) ** (1.0 / p)

    xv = fm4
    activation = jnp.einsum('bnf,fc->bnc', xv, cluster_w)
    activation = _batchnorm(activation.reshape(-1, CLUSTER_SIZE), bn1_g, bn1_b)
    activation = jax.nn.softmax(activation.reshape(-1, MAX_SAMPLES, CLUSTER_SIZE), axis=-1)
    a_sum = jnp.sum(activation, axis=-2, keepdims=True)
    a = a_sum * cluster_w2
    vlad = jnp.einsum('bnc,bnf->bfc', activation, xv)
    vlad = vlad - a
    vlad = _l2norm(vlad, axis=1)
    vlad = vlad.reshape(-1, CLUSTER_SIZE * FEATURE_SIZE)
    vlad = _l2norm(vlad, axis=1)
    vlad = _batchnorm(_hidden_matmul(vlad, hidden1_w), bn2_g, bn2_b)
    gates = jax.nn.sigmoid(_batchnorm(vlad @ gating_w, gbn_g, gbn_b))
    return (y, vlad * gates)


# speedup vs baseline: 2.1431x; 1.1669x over previous
import functools
import math

import jax
import jax.numpy as jnp
from jax import lax
from jax.experimental import pallas as pl
from jax.experimental.pallas import tpu as pltpu
from jax.experimental.pallas import tpu_sc as plsc

SUPPORT_NUM = 1
NEIGHBOR_NUM = 20
FEATURE_SIZE = 1024
MAX_SAMPLES = 256
CLUSTER_SIZE = 64
BN_EPS = 1e-5

_NW = 32


def _l2norm(x, axis):
    n = jnp.sqrt(jnp.sum(x * x, axis=axis, keepdims=True))
    return x / jnp.maximum(n, 1e-12)


def _knn_body(vr_ref, vt_ref, out_ref, *, v, n_extract, rows):
    vr = vr_ref[0]
    vt = vt_ref[0]
    inner = jnp.dot(vr, vt, preferred_element_type=jnp.float32)
    quad_r = jnp.sum(vr * vr, axis=1, keepdims=True)
    quad_t = jnp.sum(vt * vt, axis=0, keepdims=True)
    dist = -2.0 * inner + quad_r + quad_t
    iota_f = jax.lax.broadcasted_iota(
        jnp.int32, (1, v), 1).astype(jnp.float32)
    big = jnp.float32(3.0e38)
    iota_o = jax.lax.broadcasted_iota(jnp.int32, (1, 32), 1)
    acc = jnp.zeros((rows, 32), jnp.float32)
    for j in range(n_extract):
        m = jnp.min(dist, axis=1, keepdims=True)
        mi = jnp.where(dist == m, iota_f, big)
        ji = jnp.min(mi, axis=1, keepdims=True)
        dist = jnp.where(mi == ji, big, dist)
        acc = jnp.where(iota_o == j, ji, acc)
    out_ref[0] = acc.astype(jnp.int32)


def _knn_index(vertices, neighbor_num):
    bs, v, _ = vertices.shape
    rows = min(v, 256)
    vt = jnp.transpose(vertices, (0, 2, 1))
    out = pl.pallas_call(
        functools.partial(_knn_body, v=v, n_extract=neighbor_num + 1,
                          rows=rows),
        grid=(bs, v // rows),
        in_specs=[
            pl.BlockSpec((1, rows, 3), lambda b, i: (b, i, 0)),
            pl.BlockSpec((1, 3, v), lambda b, i: (b, 0, 0)),
        ],
        out_specs=pl.BlockSpec((1, rows, 32), lambda b, i: (b, i, 0)),
        out_shape=jax.ShapeDtypeStruct((bs, v, 32), jnp.int32),
    )(vertices, vt)
    return out[:, :, 1:neighbor_num + 1]


def _sc_gather(table, idx):
    t, d = table.shape
    n = idx.shape[0]
    assert d % 128 == 0 and n % (8 * _NW) == 0
    b_per_w = n // _NW
    chunk = min(b_per_w, max(8, min(128, (98304 // d) & ~7)))
    while b_per_w % chunk:
        chunk -= 8
    n_chunks = b_per_w // chunk
    mesh = plsc.VectorSubcoreMesh(core_axis_name="c", subcore_axis_name="s")

    @functools.partial(
        pl.kernel, mesh=mesh,
        out_type=jax.ShapeDtypeStruct((n, d), jnp.float32),
        scratch_types=[
            pltpu.VMEM((chunk,), jnp.int32),
            pltpu.VMEM((chunk, d), jnp.float32),
            pltpu.SemaphoreType.DMA,
        ],
        compiler_params=pltpu.CompilerParams(use_tc_tiling_on_sc=True),
    )
    def gk(table_hbm, idx_hbm, out_hbm, idx_v, rows_v, sem):
        wid = lax.axis_index("s") * 2 + lax.axis_index("c")
        base = wid * b_per_w

        def body(i, carry):
            off = base + i * chunk
            pltpu.sync_copy(idx_hbm.at[pl.ds(off, chunk)], idx_v)
            pltpu.async_copy(table_hbm.at[idx_v], rows_v, sem).wait()
            pltpu.sync_copy(rows_v, out_hbm.at[pl.ds(off, chunk)])
            return carry

        lax.fori_loop(0, n_chunks, body, 0)

    return gk(table, idx)


def _flat_nbr_idx(index, v):
    bs = index.shape[0]
    return (index + (jnp.arange(bs, dtype=index.dtype)[:, None, None] * v)
            ).reshape(-1)


def _pad128(x):
    d = x.shape[-1]
    dp = (d + 127) & ~127
    if d == dp:
        return x
    pads = [(0, 0)] * (x.ndim - 1) + [(0, dp - d)]
    return jnp.pad(x, pads)


def _dirnorm(nbr, ctr, b, n):
    d = nbr.reshape(b, n, 128) - ctr[:, None, :]
    s = jnp.sqrt(jnp.sum(d * d, axis=2, keepdims=True))
    dn = d / jnp.maximum(s, 1e-12)
    return dn.reshape(b * n, 128)


def _surface_body(nbr_ref, ctr_ref, sd_ref, out_ref, *, b, n):
    dn = _dirnorm(nbr_ref[...], ctr_ref[...], b, n)
    theta = jnp.maximum(
        jnp.dot(dn, sd_ref[...], preferred_element_type=jnp.float32), 0.0)
    out_ref[...] = jnp.max(theta.reshape(b, n, -1), axis=1)


def _conv_surface(nd_g, vtab, sd, kernel_num):
    bv = vtab.shape[0]
    n = NEIGHBOR_NUM
    b = min(bv, 512)
    sdp = jnp.zeros((128, kernel_num), jnp.float32).at[:3].set(
        _l2norm(sd, axis=0))
    return pl.pallas_call(
        functools.partial(_surface_body, b=b, n=n),
        grid=(bv // b,),
        in_specs=[
            pl.BlockSpec((b * n, 128), lambda i: (i, 0)),
            pl.BlockSpec((b, 128), lambda i: (i, 0)),
            pl.BlockSpec((128, kernel_num), lambda i: (0, 0)),
        ],
        out_specs=pl.BlockSpec((b, kernel_num), lambda i: (i, 0)),
        out_shape=jax.ShapeDtypeStruct((bv, kernel_num), jnp.float32),
    )(nd_g, vtab, sdp)


def _conv_body(nbr_ref, ctr_ref, fs_ref, fc_ref, sd_ref, out_ref, *, b, n, c):
    dn = _dirnorm(nbr_ref[...], ctr_ref[...], b, n)
    theta = jnp.maximum(
        jnp.dot(dn, sd_ref[...], preferred_element_type=jnp.float32), 0.0)
    act = theta.reshape(b, n, -1) * fs_ref[...].reshape(b, n, -1)
    red = jnp.max(act, axis=1)
    out_ref[...] = fc_ref[...] + red[:, :c]


def _conv_layer(nd_g, vtab, fs_g, fc, sd, out_channel):
    bv, c = fc.shape
    cp = fs_g.shape[1]
    n = NEIGHBOR_NUM
    b = min(bv, max(8, 65536 // cp))
    while bv % b:
        b -= 8
    sdp = jnp.zeros((128, cp), jnp.float32).at[:3, :c].set(
        _l2norm(sd, axis=0))
    return pl.pallas_call(
        functools.partial(_conv_body, b=b, n=n, c=c),
        grid=(bv // b,),
        in_specs=[
            pl.BlockSpec((b * n, 128), lambda i: (i, 0)),
            pl.BlockSpec((b, 128), lambda i: (i, 0)),
            pl.BlockSpec((b * n, cp), lambda i: (i, 0)),
            pl.BlockSpec((b, c), lambda i: (i, 0)),
            pl.BlockSpec((128, cp), lambda i: (0, 0)),
        ],
        out_specs=pl.BlockSpec((b, c), lambda i: (i, 0)),
        out_shape=jax.ShapeDtypeStruct((bv, c), jnp.float32),
    )(nd_g, vtab, fs_g, fc, sdp)


def _pool_body(g_ref, out_ref, *, b, n, c):
    out_ref[...] = jnp.max(g_ref[...].reshape(b, n, -1), axis=1)[:, :c]


def _pool_max(g, pool_rows, c):
    cp = g.shape[1]
    n = NEIGHBOR_NUM
    b = min(pool_rows, max(8, 65536 // cp))
    while pool_rows % b:
        b -= 8
    return pl.pallas_call(
        functools.partial(_pool_body, b=b, n=n, c=c),
        grid=(pool_rows // b,),
        in_specs=[pl.BlockSpec((b * n, cp), lambda i: (i, 0))],
        out_specs=pl.BlockSpec((b, c), lambda i: (i, 0)),
        out_shape=jax.ShapeDtypeStruct((pool_rows, c), jnp.float32),
    )(g)


def _hidden_mm_body(x_ref, w_ref, o_ref):
    @pl.when(pl.program_id(0) == 0)
    def _init():
        o_ref[...] = jnp.zeros_like(o_ref)

    o_ref[...] += jnp.dot(x_ref[...], w_ref[...],
                          preferred_element_type=jnp.float32)


def _hidden_matmul(xv, w):
    b = xv.shape[0]
    k, n = w.shape
    xp = jnp.zeros((8, k), xv.dtype).at[:b].set(xv)
    kb = 4096
    out = pl.pallas_call(
        _hidden_mm_body,
        grid=(k // kb,),
        in_specs=[
            pl.BlockSpec((8, kb), lambda i: (0, i)),
            pl.BlockSpec((kb, n), lambda i: (i, 0)),
        ],
        out_specs=pl.BlockSpec((8, n), lambda i: (0, 0)),
        out_shape=jax.ShapeDtypeStruct((8, n), jnp.float32),
        compiler_params=pltpu.CompilerParams(
            dimension_semantics=("arbitrary",)),
    )(xp, w)
    return out[:b]


def _batchnorm(x, gamma, beta):
    mean = jnp.mean(x, axis=0)
    var = jnp.mean((x - mean) ** 2, axis=0)
    return gamma * (x - mean) / jnp.sqrt(var + BN_EPS) + beta


def kernel(x, dirs0, w1, b1, dirs1, w2, b2, dirs2, w3, b3, dirs3, w4, b4, dirs4, cluster_w, cluster_w2, hidden1_w, bn1_g, bn1_b, bn2_g, bn2_b, gating_w, gbn_g, gbn_b, gem_p):
    bs = x.shape[0]
    vertices = x.reshape(bs, -1, 3)
    v = vertices.shape[1]
    rate = 4

    ni = _knn_index(vertices, NEIGHBOR_NUM)
    fi = _flat_nbr_idx(ni, v)
    vtab = _pad128(vertices).reshape(bs * v, 128)
    nd_g = _sc_gather(vtab, fi)
    fm0 = jax.nn.relu(_conv_surface(nd_g, vtab, dirs0, 32))
    fo1 = fm0 @ w1 + b1
    fs1_g = _sc_gather(_pad128(fo1[:, 64:]), fi)
    fm1 = jax.nn.relu(_conv_layer(nd_g, vtab, fs1_g, fo1[:, :64], dirs1, 64))

    v2 = v // rate
    samp = jnp.arange(v2) * rate
    fi_s = _flat_nbr_idx(ni[:, samp, :], v)
    p1_g = _sc_gather(_pad128(fm1), fi_s)
    fm1p = _pool_max(p1_g, bs * v2, 64)
    vertices = vertices[:, samp, :]

    ni = _knn_index(vertices, NEIGHBOR_NUM)
    fi = _flat_nbr_idx(ni, v2)
    vtab = _pad128(vertices).reshape(bs * v2, 128)
    nd_g = _sc_gather(vtab, fi)
    fo2 = fm1p @ w2 + b2
    fs2_g = _sc_gather(fo2[:, 128:], fi)
    fm2 = jax.nn.relu(_conv_layer(nd_g, vtab, fs2_g, fo2[:, :128], dirs2, 128))
    fo3 = fm2 @ w3 + b3
    fs3_g = _sc_gather(fo3[:, 256:], fi)
    fm3 = jax.nn.relu(_conv_layer(nd_g, vtab, fs3_g, fo3[:, :256], dirs3, 256))

    v3 = v2 // rate
    samp = jnp.arange(v3) * rate
    fi_s = _flat_nbr_idx(ni[:, samp, :], v2)
    p2_g = _sc_gather(fm3, fi_s)
    fm3p = _pool_max(p2_g, bs * v3, 256)
    vertices = vertices[:, samp, :]

    ni = _knn_index(vertices, NEIGHBOR_NUM)
    fi = _flat_nbr_idx(ni, v3)
    vtab = _pad128(vertices).reshape(bs * v3, 128)
    nd_g = _sc_gather(vtab, fi)
    fo4 = fm3p @ w4 + b4
    fs4_g = _sc_gather(fo4[:, 1024:], fi)
    fm4 = _conv_layer(nd_g, vtab, fs4_g, fo4[:, :1024], dirs4, 1024)
    fm4 = fm4.reshape(bs, v3, 1024)

    p = gem_p[0]
    g = jnp.maximum(fm4, 1e-06) ** p
    y = jnp.mean(g, axis=1) ** (1.0 / p)

    xv = fm4
    activation = jnp.einsum('bnf,fc->bnc', xv, cluster_w)
    activation = _batchnorm(activation.reshape(-1, CLUSTER_SIZE), bn1_g, bn1_b)
    activation = jax.nn.softmax(
        activation.reshape(-1, MAX_SAMPLES, CLUSTER_SIZE), axis=-1)
    a_sum = jnp.sum(activation, axis=-2, keepdims=True)
    a = a_sum * cluster_w2
    vlad = jnp.einsum('bnc,bnf->bfc', activation, xv)
    vlad = vlad - a
    vlad = _l2norm(vlad, axis=1)
    vlad = vlad.reshape(-1, CLUSTER_SIZE * FEATURE_SIZE)
    vlad = _l2norm(vlad, axis=1)
    vlad = _batchnorm(_hidden_matmul(vlad, hidden1_w), bn2_g, bn2_b)
    gates = jax.nn.sigmoid(_batchnorm(vlad @ gating_w, gbn_g, gbn_b))
    return (y, vlad * gates)


# speedup vs baseline: 2.2031x; 1.0280x over previous
import functools
import math

import jax
import jax.numpy as jnp
from jax import lax
from jax.experimental import pallas as pl
from jax.experimental.pallas import tpu as pltpu
from jax.experimental.pallas import tpu_sc as plsc

SUPPORT_NUM = 1
NEIGHBOR_NUM = 20
FEATURE_SIZE = 1024
MAX_SAMPLES = 256
CLUSTER_SIZE = 64
BN_EPS = 1e-5

_NW = 32


def _l2norm(x, axis):
    n = jnp.sqrt(jnp.sum(x * x, axis=axis, keepdims=True))
    return x / jnp.maximum(n, 1e-12)


def _knn_body(vr_ref, vt_ref, out_ref, *, v, n_extract, rows):
    vr = vr_ref[0]
    vt = vt_ref[0]
    inner = jnp.dot(vr, vt, preferred_element_type=jnp.float32)
    quad_r = jnp.sum(vr * vr, axis=1, keepdims=True)
    quad_t = jnp.sum(vt * vt, axis=0, keepdims=True)
    dist = -2.0 * inner + quad_r + quad_t
    iota_f = jax.lax.broadcasted_iota(
        jnp.int32, (1, v), 1).astype(jnp.float32)
    big = jnp.float32(3.0e38)
    iota_o = jax.lax.broadcasted_iota(jnp.int32, (1, 32), 1)
    acc = jnp.zeros((rows, 32), jnp.float32)
    for j in range(n_extract):
        m = jnp.min(dist, axis=1, keepdims=True)
        mi = jnp.where(dist == m, iota_f, big)
        ji = jnp.min(mi, axis=1, keepdims=True)
        dist = jnp.where(mi == ji, big, dist)
        acc = jnp.where(iota_o == j, ji, acc)
    out_ref[0] = acc.astype(jnp.int32)


def _knn_index(vertices, neighbor_num):
    bs, v, _ = vertices.shape
    rows = min(v, 256)
    vt = jnp.transpose(vertices, (0, 2, 1))
    out = pl.pallas_call(
        functools.partial(_knn_body, v=v, n_extract=neighbor_num + 1,
                          rows=rows),
        grid=(bs, v // rows),
        in_specs=[
            pl.BlockSpec((1, rows, 3), lambda b, i: (b, i, 0)),
            pl.BlockSpec((1, 3, v), lambda b, i: (b, 0, 0)),
        ],
        out_specs=pl.BlockSpec((1, rows, 32), lambda b, i: (b, i, 0)),
        out_shape=jax.ShapeDtypeStruct((bs, v, 32), jnp.int32),
    )(vertices, vt)
    return out[:, :, 1:neighbor_num + 1]


def _sc_gather(table, idx):
    t, d = table.shape
    n = idx.shape[0]
    assert d % 128 == 0 and n % (8 * _NW) == 0
    b_per_w = n // _NW
    chunk = min(b_per_w, max(8, min(128, (98304 // d) & ~7)))
    while b_per_w % chunk:
        chunk -= 8
    n_chunks = b_per_w // chunk
    mesh = plsc.VectorSubcoreMesh(core_axis_name="c", subcore_axis_name="s")

    @functools.partial(
        pl.kernel, mesh=mesh,
        out_type=jax.ShapeDtypeStruct((n, d), jnp.float32),
        scratch_types=[
            pltpu.VMEM((chunk,), jnp.int32),
            pltpu.VMEM((chunk, d), jnp.float32),
            pltpu.SemaphoreType.DMA,
        ],
        compiler_params=pltpu.CompilerParams(use_tc_tiling_on_sc=True),
    )
    def gk(table_hbm, idx_hbm, out_hbm, idx_v, rows_v, sem):
        wid = lax.axis_index("s") * 2 + lax.axis_index("c")
        base = wid * b_per_w

        def body(i, carry):
            off = base + i * chunk
            pltpu.sync_copy(idx_hbm.at[pl.ds(off, chunk)], idx_v)
            pltpu.async_copy(table_hbm.at[idx_v], rows_v, sem).wait()
            pltpu.sync_copy(rows_v, out_hbm.at[pl.ds(off, chunk)])
            return carry

        lax.fori_loop(0, n_chunks, body, 0)

    return gk(table, idx)


def _flat_nbr_idx(index, v):
    bs = index.shape[0]
    return (index + (jnp.arange(bs, dtype=index.dtype)[:, None, None] * v)
            ).reshape(-1)


def _pad128(x):
    d = x.shape[-1]
    dp = (d + 127) & ~127
    if d == dp:
        return x
    pads = [(0, 0)] * (x.ndim - 1) + [(0, dp - d)]
    return jnp.pad(x, pads)


def _dir_theta(nbr, ctr, sd, b, n):
    d = (nbr.reshape(b, n, 128) - ctr[:, None, :]).reshape(b * n, 128)
    s = jnp.sqrt(jnp.sum(d * d, axis=1, keepdims=True))
    rinv = 1.0 / jnp.maximum(s, 1e-12)
    t = jnp.maximum(jnp.dot(d, sd, preferred_element_type=jnp.float32), 0.0)
    return t * rinv


def _surface_body(nbr_ref, ctr_ref, sd_ref, out_ref, *, b, n):
    theta = _dir_theta(nbr_ref[...], ctr_ref[...], sd_ref[...], b, n)
    out_ref[...] = jnp.max(theta.reshape(b, n, -1), axis=1)


def _conv_surface(nd_g, vtab, sd, kernel_num):
    bv = vtab.shape[0]
    n = NEIGHBOR_NUM
    b = min(bv, 512)
    sdp = jnp.zeros((128, kernel_num), jnp.float32).at[:3].set(
        _l2norm(sd, axis=0))
    return pl.pallas_call(
        functools.partial(_surface_body, b=b, n=n),
        grid=(bv // b,),
        in_specs=[
            pl.BlockSpec((b * n, 128), lambda i: (i, 0)),
            pl.BlockSpec((b, 128), lambda i: (i, 0)),
            pl.BlockSpec((128, kernel_num), lambda i: (0, 0)),
        ],
        out_specs=pl.BlockSpec((b, kernel_num), lambda i: (i, 0)),
        out_shape=jax.ShapeDtypeStruct((bv, kernel_num), jnp.float32),
    )(nd_g, vtab, sdp)


def _conv_body(nbr_ref, ctr_ref, fs_ref, fc_ref, sd_ref, out_ref, *, b, n, c):
    theta = _dir_theta(nbr_ref[...], ctr_ref[...], sd_ref[...], b, n)
    act = theta.reshape(b, n, -1) * fs_ref[...].reshape(b, n, -1)
    red = jnp.max(act, axis=1)
    out_ref[...] = fc_ref[...] + red[:, :c]


def _conv_layer(nd_g, vtab, fs_g, fc, sd, out_channel):
    bv, c = fc.shape
    cp = fs_g.shape[1]
    n = NEIGHBOR_NUM
    b = min(bv, max(8, 65536 // cp))
    while bv % b:
        b -= 8
    sdp = jnp.zeros((128, cp), jnp.float32).at[:3, :c].set(
        _l2norm(sd, axis=0))
    return pl.pallas_call(
        functools.partial(_conv_body, b=b, n=n, c=c),
        grid=(bv // b,),
        in_specs=[
            pl.BlockSpec((b * n, 128), lambda i: (i, 0)),
            pl.BlockSpec((b, 128), lambda i: (i, 0)),
            pl.BlockSpec((b * n, cp), lambda i: (i, 0)),
            pl.BlockSpec((b, c), lambda i: (i, 0)),
            pl.BlockSpec((128, cp), lambda i: (0, 0)),
        ],
        out_specs=pl.BlockSpec((b, c), lambda i: (i, 0)),
        out_shape=jax.ShapeDtypeStruct((bv, c), jnp.float32),
    )(nd_g, vtab, fs_g, fc, sdp)


def _pool_body(g_ref, out_ref, *, b, n, c):
    out_ref[...] = jnp.max(g_ref[...].reshape(b, n, -1), axis=1)[:, :c]


def _pool_max(g, pool_rows, c):
    cp = g.shape[1]
    n = NEIGHBOR_NUM
    b = min(pool_rows, max(8, 65536 // cp))
    while pool_rows % b:
        b -= 8
    return pl.pallas_call(
        functools.partial(_pool_body, b=b, n=n, c=c),
        grid=(pool_rows // b,),
        in_specs=[pl.BlockSpec((b * n, cp), lambda i: (i, 0))],
        out_specs=pl.BlockSpec((b, c), lambda i: (i, 0)),
        out_shape=jax.ShapeDtypeStruct((pool_rows, c), jnp.float32),
    )(g)


def _hidden_mm_body(x_ref, w_ref, o_ref):
    @pl.when(pl.program_id(0) == 0)
    def _init():
        o_ref[...] = jnp.zeros_like(o_ref)

    o_ref[...] += jnp.dot(x_ref[...], w_ref[...],
                          preferred_element_type=jnp.float32)


def _hidden_matmul(xv, w):
    b = xv.shape[0]
    k, n = w.shape
    xp = jnp.zeros((8, k), xv.dtype).at[:b].set(xv)
    kb = 4096
    out = pl.pallas_call(
        _hidden_mm_body,
        grid=(k // kb,),
        in_specs=[
            pl.BlockSpec((8, kb), lambda i: (0, i)),
            pl.BlockSpec((kb, n), lambda i: (i, 0)),
        ],
        out_specs=pl.BlockSpec((8, n), lambda i: (0, 0)),
        out_shape=jax.ShapeDtypeStruct((8, n), jnp.float32),
        compiler_params=pltpu.CompilerParams(
            dimension_semantics=("arbitrary",)),
    )(xp, w)
    return out[:b]


def _batchnorm(x, gamma, beta):
    mean = jnp.mean(x, axis=0)
    var = jnp.mean((x - mean) ** 2, axis=0)
    return gamma * (x - mean) / jnp.sqrt(var + BN_EPS) + beta


def kernel(x, dirs0, w1, b1, dirs1, w2, b2, dirs2, w3, b3, dirs3, w4, b4, dirs4, cluster_w, cluster_w2, hidden1_w, bn1_g, bn1_b, bn2_g, bn2_b, gating_w, gbn_g, gbn_b, gem_p):
    bs = x.shape[0]
    vertices = x.reshape(bs, -1, 3)
    v = vertices.shape[1]
    rate = 4

    ni = _knn_index(vertices, NEIGHBOR_NUM)
    fi = _flat_nbr_idx(ni, v)
    vtab = _pad128(vertices).reshape(bs * v, 128)
    nd_g = _sc_gather(vtab, fi)
    fm0 = jax.nn.relu(_conv_surface(nd_g, vtab, dirs0, 32))
    fo1 = fm0 @ w1 + b1
    fs1_g = _sc_gather(_pad128(fo1[:, 64:]), fi)
    fm1 = jax.nn.relu(_conv_layer(nd_g, vtab, fs1_g, fo1[:, :64], dirs1, 64))

    v2 = v // rate
    samp = jnp.arange(v2) * rate
    fi_s = _flat_nbr_idx(ni[:, samp, :], v)
    p1_g = _sc_gather(_pad128(fm1), fi_s)
    fm1p = _pool_max(p1_g, bs * v2, 64)
    vertices = vertices[:, samp, :]

    ni = _knn_index(vertices, NEIGHBOR_NUM)
    fi = _flat_nbr_idx(ni, v2)
    vtab = _pad128(vertices).reshape(bs * v2, 128)
    nd_g = _sc_gather(vtab, fi)
    fo2 = fm1p @ w2 + b2
    fs2_g = _sc_gather(fo2[:, 128:], fi)
    fm2 = jax.nn.relu(_conv_layer(nd_g, vtab, fs2_g, fo2[:, :128], dirs2, 128))
    fo3 = fm2 @ w3 + b3
    fs3_g = _sc_gather(fo3[:, 256:], fi)
    fm3 = jax.nn.relu(_conv_layer(nd_g, vtab, fs3_g, fo3[:, :256], dirs3, 256))

    v3 = v2 // rate
    samp = jnp.arange(v3) * rate
    fi_s = _flat_nbr_idx(ni[:, samp, :], v2)
    p2_g = _sc_gather(fm3, fi_s)
    fm3p = _pool_max(p2_g, bs * v3, 256)
    vertices = vertices[:, samp, :]

    ni = _knn_index(vertices, NEIGHBOR_NUM)
    fi = _flat_nbr_idx(ni, v3)
    vtab = _pad128(vertices).reshape(bs * v3, 128)
    nd_g = _sc_gather(vtab, fi)
    fo4 = fm3p @ w4 + b4
    fs4_g = _sc_gather(fo4[:, 1024:], fi)
    fm4 = _conv_layer(nd_g, vtab, fs4_g, fo4[:, :1024], dirs4, 1024)
    fm4 = fm4.reshape(bs, v3, 1024)

    p = gem_p[0]
    g = jnp.maximum(fm4, 1e-06) ** p
    y = jnp.mean(g, axis=1) ** (1.0 / p)

    xv = fm4
    activation = jnp.einsum('bnf,fc->bnc', xv, cluster_w)
    activation = _batchnorm(activation.reshape(-1, CLUSTER_SIZE), bn1_g, bn1_b)
    activation = jax.nn.softmax(
        activation.reshape(-1, MAX_SAMPLES, CLUSTER_SIZE), axis=-1)
    a_sum = jnp.sum(activation, axis=-2, keepdims=True)
    a = a_sum * cluster_w2
    vlad = jnp.einsum('bnc,bnf->bfc', activation, xv)
    vlad = vlad - a
    vlad = _l2norm(vlad, axis=1)
    vlad = vlad.reshape(-1, CLUSTER_SIZE * FEATURE_SIZE)
    vlad = _l2norm(vlad, axis=1)
    vlad = _batchnorm(_hidden_matmul(vlad, hidden1_w), bn2_g, bn2_b)
    gates = jax.nn.sigmoid(_batchnorm(vlad @ gating_w, gbn_g, gbn_b))
    return (y, vlad * gates)


# speedup vs baseline: 2.3149x; 1.0508x over previous
import functools
import math

import jax
import jax.numpy as jnp
from jax import lax
from jax.experimental import pallas as pl
from jax.experimental.pallas import tpu as pltpu
from jax.experimental.pallas import tpu_sc as plsc

SUPPORT_NUM = 1
NEIGHBOR_NUM = 20
FEATURE_SIZE = 1024
MAX_SAMPLES = 256
CLUSTER_SIZE = 64
BN_EPS = 1e-5

_NW = 32


def _l2norm(x, axis):
    n = jnp.sqrt(jnp.sum(x * x, axis=axis, keepdims=True))
    return x / jnp.maximum(n, 1e-12)


def _knn_body(vr_ref, vt_ref, out_ref, *, v, n_extract, rows):
    vr = vr_ref[0]
    vt = vt_ref[0]
    inner = jnp.dot(vr, vt, preferred_element_type=jnp.float32)
    quad_r = jnp.sum(vr * vr, axis=1, keepdims=True)
    quad_t = jnp.sum(vt * vt, axis=0, keepdims=True)
    dist = -2.0 * inner + quad_r + quad_t
    iota_f = jax.lax.broadcasted_iota(
        jnp.int32, (1, v), 1).astype(jnp.float32)
    big = jnp.float32(3.0e38)
    iota_o = jax.lax.broadcasted_iota(jnp.int32, (1, 32), 1)
    acc = jnp.zeros((rows, 32), jnp.float32)
    for j in range(n_extract):
        m = jnp.min(dist, axis=1, keepdims=True)
        mi = jnp.where(dist == m, iota_f, big)
        ji = jnp.min(mi, axis=1, keepdims=True)
        dist = jnp.where(mi == ji, big, dist)
        acc = jnp.where(iota_o == j, ji, acc)
    out_ref[0] = acc.astype(jnp.int32)


def _knn_index(vertices, neighbor_num):
    bs, v, _ = vertices.shape
    rows = min(v, 256)
    vt = jnp.transpose(vertices, (0, 2, 1))
    out = pl.pallas_call(
        functools.partial(_knn_body, v=v, n_extract=neighbor_num + 1,
                          rows=rows),
        grid=(bs, v // rows),
        in_specs=[
            pl.BlockSpec((1, rows, 3), lambda b, i: (b, i, 0)),
            pl.BlockSpec((1, 3, v), lambda b, i: (b, 0, 0)),
        ],
        out_specs=pl.BlockSpec((1, rows, 32), lambda b, i: (b, i, 0)),
        out_shape=jax.ShapeDtypeStruct((bs, v, 32), jnp.int32),
    )(vertices, vt)
    return out[:, :, 1:neighbor_num + 1]


def _sc_gather(table, idx):
    t, d = table.shape
    n = idx.shape[0]
    assert d % 128 == 0 and n % (8 * _NW) == 0
    b_per_w = n // _NW
    chunk = min(b_per_w, max(8, min(128, (49152 // d) & ~7)))
    while b_per_w % chunk or (b_per_w // chunk) % 2:
        chunk -= 8
    n_chunks = b_per_w // chunk
    n_pairs = n_chunks // 2
    mesh = plsc.VectorSubcoreMesh(core_axis_name="c", subcore_axis_name="s")

    @functools.partial(
        pl.kernel, mesh=mesh,
        out_type=jax.ShapeDtypeStruct((n, d), jnp.float32),
        scratch_types=[
            pltpu.VMEM((chunk,), jnp.int32),
            pltpu.VMEM((chunk,), jnp.int32),
            pltpu.VMEM((chunk, d), jnp.float32),
            pltpu.VMEM((chunk, d), jnp.float32),
            pltpu.SemaphoreType.DMA,
            pltpu.SemaphoreType.DMA,
        ],
        compiler_params=pltpu.CompilerParams(use_tc_tiling_on_sc=True),
    )
    def gk(table_hbm, idx_hbm, out_hbm, idx0, idx1, rows0, rows1, sem0, sem1):
        wid = lax.axis_index("s") * 2 + lax.axis_index("c")
        base = wid * b_per_w

        def start(idxbuf, rowsbuf, sem, off):
            pltpu.sync_copy(idx_hbm.at[pl.ds(off, chunk)], idxbuf)
            pltpu.async_copy(table_hbm.at[idxbuf], rowsbuf, sem)

        def finish(idxbuf, rowsbuf, sem, off):
            pltpu.make_async_copy(
                table_hbm.at[idxbuf], rowsbuf, sem).wait()
            pltpu.sync_copy(rowsbuf, out_hbm.at[pl.ds(off, chunk)])

        start(idx0, rows0, sem0, base)

        def body(g, carry):
            o1 = base + (2 * g + 1) * chunk
            start(idx1, rows1, sem1, o1)
            finish(idx0, rows0, sem0, base + 2 * g * chunk)

            @pl.when(g + 1 < n_pairs)
            def _():
                start(idx0, rows0, sem0, base + (2 * g + 2) * chunk)

            finish(idx1, rows1, sem1, o1)
            return carry

        lax.fori_loop(0, n_pairs, body, 0)

    return gk(table, idx)


def _flat_nbr_idx(index, v):
    bs = index.shape[0]
    return (index + (jnp.arange(bs, dtype=index.dtype)[:, None, None] * v)
            ).reshape(-1)


def _pad128(x):
    d = x.shape[-1]
    dp = (d + 127) & ~127
    if d == dp:
        return x
    pads = [(0, 0)] * (x.ndim - 1) + [(0, dp - d)]
    return jnp.pad(x, pads)


def _dirnorm(nbr, ctr, b, n):
    d = nbr.reshape(b, n, 128) - ctr[:, None, :]
    s = jnp.sqrt(jnp.sum(d * d, axis=2, keepdims=True))
    dn = d / jnp.maximum(s, 1e-12)
    return dn.reshape(b * n, 128)


def _surface_body(nbr_ref, ctr_ref, sd_ref, out_ref, *, b, n):
    dn = _dirnorm(nbr_ref[...], ctr_ref[...], b, n)
    theta = jnp.maximum(
        jnp.dot(dn, sd_ref[...], preferred_element_type=jnp.float32), 0.0)
    out_ref[...] = jnp.max(theta.reshape(b, n, -1), axis=1)


def _conv_surface(nd_g, vtab, sd, kernel_num):
    bv = vtab.shape[0]
    n = NEIGHBOR_NUM
    b = min(bv, 512)
    sdp = jnp.zeros((128, kernel_num), jnp.float32).at[:3].set(
        _l2norm(sd, axis=0))
    return pl.pallas_call(
        functools.partial(_surface_body, b=b, n=n),
        grid=(bv // b,),
        in_specs=[
            pl.BlockSpec((b * n, 128), lambda i: (i, 0)),
            pl.BlockSpec((b, 128), lambda i: (i, 0)),
            pl.BlockSpec((128, kernel_num), lambda i: (0, 0)),
        ],
        out_specs=pl.BlockSpec((b, kernel_num), lambda i: (i, 0)),
        out_shape=jax.ShapeDtypeStruct((bv, kernel_num), jnp.float32),
    )(nd_g, vtab, sdp)


def _conv_body(nbr_ref, ctr_ref, fs_ref, fc_ref, sd_ref, out_ref, *, b, n, c):
    dn = _dirnorm(nbr_ref[...], ctr_ref[...], b, n)
    theta = jnp.maximum(
        jnp.dot(dn, sd_ref[...], preferred_element_type=jnp.float32), 0.0)
    act = theta.reshape(b, n, -1) * fs_ref[...].reshape(b, n, -1)
    red = jnp.max(act, axis=1)
    out_ref[...] = fc_ref[...] + red[:, :c]


def _conv_layer(nd_g, vtab, fs_g, fc, sd, out_channel):
    bv, c = fc.shape
    cp = fs_g.shape[1]
    n = NEIGHBOR_NUM
    b = min(bv, max(8, 65536 // cp))
    while bv % b:
        b -= 8
    sdp = jnp.zeros((128, cp), jnp.float32).at[:3, :c].set(
        _l2norm(sd, axis=0))
    return pl.pallas_call(
        functools.partial(_conv_body, b=b, n=n, c=c),
        grid=(bv // b,),
        in_specs=[
            pl.BlockSpec((b * n, 128), lambda i: (i, 0)),
            pl.BlockSpec((b, 128), lambda i: (i, 0)),
            pl.BlockSpec((b * n, cp), lambda i: (i, 0)),
            pl.BlockSpec((b, c), lambda i: (i, 0)),
            pl.BlockSpec((128, cp), lambda i: (0, 0)),
        ],
        out_specs=pl.BlockSpec((b, c), lambda i: (i, 0)),
        out_shape=jax.ShapeDtypeStruct((bv, c), jnp.float32),
    )(nd_g, vtab, fs_g, fc, sdp)


def _pool_body(g_ref, out_ref, *, b, n, c):
    out_ref[...] = jnp.max(g_ref[...].reshape(b, n, -1), axis=1)[:, :c]


def _pool_max(g, pool_rows, c):
    cp = g.shape[1]
    n = NEIGHBOR_NUM
    b = min(pool_rows, max(8, 65536 // cp))
    while pool_rows % b:
        b -= 8
    return pl.pallas_call(
        functools.partial(_pool_body, b=b, n=n, c=c),
        grid=(pool_rows // b,),
        in_specs=[pl.BlockSpec((b * n, cp), lambda i: (i, 0))],
        out_specs=pl.BlockSpec((b, c), lambda i: (i, 0)),
        out_shape=jax.ShapeDtypeStruct((pool_rows, c), jnp.float32),
    )(g)


def _hidden_mm_body(x_ref, w_ref, o_ref):
    @pl.when(pl.program_id(0) == 0)
    def _init():
        o_ref[...] = jnp.zeros_like(o_ref)

    o_ref[...] += jnp.dot(x_ref[...], w_ref[...],
                          preferred_element_type=jnp.float32)


def _hidden_matmul(xv, w):
    b = xv.shape[0]
    k, n = w.shape
    xp = jnp.zeros((8, k), xv.dtype).at[:b].set(xv)
    kb = 4096
    out = pl.pallas_call(
        _hidden_mm_body,
        grid=(k // kb,),
        in_specs=[
            pl.BlockSpec((8, kb), lambda i: (0, i)),
            pl.BlockSpec((kb, n), lambda i: (i, 0)),
        ],
        out_specs=pl.BlockSpec((8, n), lambda i: (0, 0)),
        out_shape=jax.ShapeDtypeStruct((8, n), jnp.float32),
        compiler_params=pltpu.CompilerParams(
            dimension_semantics=("arbitrary",)),
    )(xp, w)
    return out[:b]


def _batchnorm(x, gamma, beta):
    mean = jnp.mean(x, axis=0)
    var = jnp.mean((x - mean) ** 2, axis=0)
    return gamma * (x - mean) / jnp.sqrt(var + BN_EPS) + beta


def kernel(x, dirs0, w1, b1, dirs1, w2, b2, dirs2, w3, b3, dirs3, w4, b4, dirs4, cluster_w, cluster_w2, hidden1_w, bn1_g, bn1_b, bn2_g, bn2_b, gating_w, gbn_g, gbn_b, gem_p):
    bs = x.shape[0]
    vertices = x.reshape(bs, -1, 3)
    v = vertices.shape[1]
    rate = 4

    ni = _knn_index(vertices, NEIGHBOR_NUM)
    fi = _flat_nbr_idx(ni, v)
    vtab = _pad128(vertices).reshape(bs * v, 128)
    nd_g = _sc_gather(vtab, fi)
    fm0 = jax.nn.relu(_conv_surface(nd_g, vtab, dirs0, 32))
    fo1 = fm0 @ w1 + b1
    fs1_g = _sc_gather(_pad128(fo1[:, 64:]), fi)
    fm1 = jax.nn.relu(_conv_layer(nd_g, vtab, fs1_g, fo1[:, :64], dirs1, 64))

    v2 = v // rate
    samp = jnp.arange(v2) * rate
    fi_s = _flat_nbr_idx(ni[:, samp, :], v)
    p1_g = _sc_gather(_pad128(fm1), fi_s)
    fm1p = _pool_max(p1_g, bs * v2, 64)
    vertices = vertices[:, samp, :]

    ni = _knn_index(vertices, NEIGHBOR_NUM)
    fi = _flat_nbr_idx(ni, v2)
    vtab = _pad128(vertices).reshape(bs * v2, 128)
    nd_g = _sc_gather(vtab, fi)
    fo2 = fm1p @ w2 + b2
    fs2_g = _sc_gather(fo2[:, 128:], fi)
    fm2 = jax.nn.relu(_conv_layer(nd_g, vtab, fs2_g, fo2[:, :128], dirs2, 128))
    fo3 = fm2 @ w3 + b3
    fs3_g = _sc_gather(fo3[:, 256:], fi)
    fm3 = jax.nn.relu(_conv_layer(nd_g, vtab, fs3_g, fo3[:, :256], dirs3, 256))

    v3 = v2 // rate
    samp = jnp.arange(v3) * rate
    fi_s = _flat_nbr_idx(ni[:, samp, :], v2)
    p2_g = _sc_gather(fm3, fi_s)
    fm3p = _pool_max(p2_g, bs * v3, 256)
    vertices = vertices[:, samp, :]

    ni = _knn_index(vertices, NEIGHBOR_NUM)
    fi = _flat_nbr_idx(ni, v3)
    vtab = _pad128(vertices).reshape(bs * v3, 128)
    nd_g = _sc_gather(vtab, fi)
    fo4 = fm3p @ w4 + b4
    fs4_g = _sc_gather(fo4[:, 1024:], fi)
    fm4 = _conv_layer(nd_g, vtab, fs4_g, fo4[:, :1024], dirs4, 1024)
    fm4 = fm4.reshape(bs, v3, 1024)

    p = gem_p[0]
    g = jnp.maximum(fm4, 1e-06) ** p
    y = jnp.mean(g, axis=1) ** (1.0 / p)

    xv = fm4
    activation = jnp.einsum('bnf,fc->bnc', xv, cluster_w)
    activation = _batchnorm(activation.reshape(-1, CLUSTER_SIZE), bn1_g, bn1_b)
    activation = jax.nn.softmax(
        activation.reshape(-1, MAX_SAMPLES, CLUSTER_SIZE), axis=-1)
    a_sum = jnp.sum(activation, axis=-2, keepdims=True)
    a = a_sum * cluster_w2
    vlad = jnp.einsum('bnc,bnf->bfc', activation, xv)
    vlad = vlad - a
    vlad = _l2norm(vlad, axis=1)
    vlad = vlad.reshape(-1, CLUSTER_SIZE * FEATURE_SIZE)
    vlad = _l2norm(vlad, axis=1)
    vlad = _batchnorm(_hidden_matmul(vlad, hidden1_w), bn2_g, bn2_b)
    gates = jax.nn.sigmoid(_batchnorm(vlad @ gating_w, gbn_g, gbn_b))
    return (y, vlad * gates)
